# baseline (speedup 1.0000x reference)
import functools

import jax
import jax.numpy as jnp
import numpy as np
from jax import lax
from jax.experimental import pallas as pl
from jax.experimental.pallas import tpu as pltpu
from jax.experimental.pallas import tpu_sc as plsc

_NC = 2
_NS = 16
_NW = _NC * _NS
_L = 16
_CHUNK = 128
_FLUSH = 80


def _sc_spmm_body(x_hbm, cv_f, rows_f, out_hbm,
                  b0, b1, b2, c0, c1, c2, r0, r1, r2,
                  acc_sh,
                  gs0, gs1, gs2, ss0, ss1, ss2, cs0, cs1, cs2,
                  rs0, rs1, rs2,
                  *, n_rows, n_main, n_extra):
    cid = lax.axis_index("c")
    sid = lax.axis_index("s")
    wid = sid * _NC + cid

    d = x_hbm.shape[1]
    buf = (b0, b1, b2)
    cbuf = (c0, c1, c2)
    rbuf = (r0, r1, r2)
    gsem = (gs0, gs1, gs2)
    ssem = (ss0, ss1, ss2)
    csem = (cs0, cs1, cs2)
    rsem = (rs0, rs1, rs2)

    n_units = n_rows // _FLUSH
    per = n_units // _NS
    hi = n_units % _NS
    ubase = sid * per + jnp.minimum(sid, hi)
    ucnt = per + jnp.where(sid < hi, 1, 0)

    def chunk_id(c):
        return wid * n_main + c

    def start_cv(s, c):
        pltpu.async_copy(cv_f.at[chunk_id(c)], cbuf[s], csem[s])

    def wait_cv(s, c):
        pltpu.make_async_copy(cv_f.at[chunk_id(c)], cbuf[s], csem[s]).wait()

    def start_rows(s, c):
        pltpu.async_copy(rows_f.at[chunk_id(c)], rbuf[s], rsem[s])

    def wait_rows(s, c):
        pltpu.make_async_copy(rows_f.at[chunk_id(c)], rbuf[s],
                              rsem[s]).wait()

    _H = _CHUNK // 2

    def start_gather(s):
        pltpu.async_copy(x_hbm.at[cbuf[s].at[0, pl.ds(0, _H)]],
                         buf[s].at[pl.ds(0, _H)], gsem[s])
        pltpu.async_copy(x_hbm.at[cbuf[s].at[0, pl.ds(_H, _H)]],
                         buf[s].at[pl.ds(_H, _H)], gsem[s])

    def wait_gather_half(s, h):
        pltpu.make_async_copy(x_hbm.at[cbuf[s].at[0, pl.ds(h * _H, _H)]],
                              buf[s].at[pl.ds(h * _H, _H)], gsem[s]).wait()

    def wait_gather(s):
        wait_gather_half(s, 0)
        wait_gather_half(s, 1)

    def start_scatter(s):
        pltpu.async_copy(buf[s], acc_sh.at[rbuf[s].at[0]], ssem[s], add=True)

    def wait_scatter(s):
        pltpu.make_async_copy(buf[s], acc_sh.at[rbuf[s].at[0]],
                              ssem[s]).wait()

    def _scale_half(b, vref, h):
        @pl.loop(h * (_H // _L), (h + 1) * (_H // _L))
        def _(g):
            v16i = vref[0, pl.ds(_CHUNK + g * _L, _L)]
            v16 = lax.bitcast_convert_type(v16i, jnp.float32)

            @pl.loop(0, _L, unroll=4)
            def _(t):
                vb = v16.at[jnp.full((_L,), t, jnp.int32)].get(
                    mode="promise_in_bounds")
                e = g * _L + t
                for j in range(d // _L):
                    sl = pl.ds(j * _L, _L)
                    b[e, sl] = b[e, sl] * vb

    n3 = (n_main // 3) * 3

    if n3 >= 3:
        start_cv(0, 0)
        start_rows(0, 0)
        start_cv(1, 1)
        start_rows(1, 1)
        start_cv(2, 2)
        wait_cv(0, 0)
        start_gather(0)
        wait_cv(1, 1)
        start_gather(1)

    @pl.loop(0, _FLUSH)
    def _zero_buf(r):
        for j in range(d // _L):
            b2[r, pl.ds(j * _L, _L)] = jnp.zeros((_L,), jnp.float32)

    @pl.loop(ubase, ubase + ucnt)
    def _zero(u):
        off = pl.multiple_of(u * _FLUSH, 8)
        pltpu.sync_copy(b2.at[pl.ds(0, _FLUSH)], acc_sh.at[pl.ds(off, _FLUSH)])
    plsc.subcore_barrier()

    if n3 >= 3:
        @pl.loop(0, n3, step=3)
        def _main(c):
            for k in range(3):
                cc = c + k
                s = k
                s2 = (k + 2) % 3

                @pl.when(cc >= 1)
                def _():
                    wait_scatter(s2)

                @pl.when(cc + 2 < n3)
                def _():
                    start_rows(s2, cc + 2)
                    wait_cv(s2, cc + 2)
                    start_gather(s2)

                wait_gather_half(s, 0)
                _scale_half(buf[s], cbuf[s], 0)
                wait_gather_half(s, 1)
                _scale_half(buf[s], cbuf[s], 1)

                @pl.when(cc + 3 < n3)
                def _():
                    start_cv(s, cc + 3)

                wait_rows(s, cc)
                start_scatter(s)

        wait_scatter((n3 - 1) % 3)

    for cc in range(n3, n_main):
        start_cv(0, cc)
        start_rows(0, cc)
        wait_cv(0, cc)
        start_gather(0)
        wait_gather(0)
        _scale_half(b0, c0, 0); _scale_half(b0, c0, 1)
        wait_rows(0, cc)
        start_scatter(0)
        wait_scatter(0)

    if n_extra:
        @pl.when(wid < n_extra)
        def _tail():
            ct = _NW * n_main + wid
            pltpu.async_copy(cv_f.at[ct], c0, cs0)
            pltpu.async_copy(rows_f.at[ct], r0, rs0)
            pltpu.make_async_copy(cv_f.at[ct], c0, cs0).wait()
            pltpu.make_async_copy(rows_f.at[ct], r0, rs0).wait()
            pltpu.sync_copy(x_hbm.at[c0.at[0, pl.ds(0, _CHUNK)]], b0)
            _scale_half(b0, c0, 0); _scale_half(b0, c0, 1)
            pltpu.sync_copy(b0, acc_sh.at[r0.at[0]], add=True)

    plsc.subcore_barrier()

    @pl.loop(ubase, ubase + ucnt)
    def _flush(u):
        off = pl.multiple_of(u * _FLUSH, 8)
        pltpu.sync_copy(acc_sh.at[pl.ds(off, _FLUSH)],
                        out_hbm.at[cid, pl.ds(off, _FLUSH)])


def _sc_spmm(x, cv_f, rows_f, n_main, n_extra):
    n_rows, d = x.shape

    body = functools.partial(
        _sc_spmm_body, n_rows=n_rows, n_main=n_main, n_extra=n_extra)
    dma = pltpu.SemaphoreType.DMA
    return pl.kernel(
        body,
        out_type=jax.ShapeDtypeStruct((_NC, n_rows, d), jnp.float32),
        mesh=plsc.VectorSubcoreMesh(core_axis_name="c", subcore_axis_name="s"),
        scratch_types=[
            pltpu.VMEM((_CHUNK, d), jnp.float32),
            pltpu.VMEM((_CHUNK, d), jnp.float32),
            pltpu.VMEM((_CHUNK, d), jnp.float32),
            pltpu.VMEM((1, 2 * _CHUNK), jnp.int32),
            pltpu.VMEM((1, 2 * _CHUNK), jnp.int32),
            pltpu.VMEM((1, 2 * _CHUNK), jnp.int32),
            pltpu.VMEM((1, _CHUNK), jnp.int32),
            pltpu.VMEM((1, _CHUNK), jnp.int32),
            pltpu.VMEM((1, _CHUNK), jnp.int32),
            pltpu.VMEM_SHARED((n_rows, d), jnp.float32),
            dma, dma, dma,
            dma, dma, dma,
            dma, dma, dma,
            dma, dma, dma,
        ],
    )(x, cv_f, rows_f)


def _epilogue_body(w_ref, acc_ref, o_ref):
    a = acc_ref[0] + acc_ref[1]
    s = a * w_ref[0]
    n2 = jnp.sum(s * s, axis=1, keepdims=True)
    y = s * lax.rsqrt(jnp.maximum(n2, 1e-24))
    o_ref[...] = 0.5 * y * (1.0 + lax.erf(y * np.float32(1.0 / np.sqrt(2.0))))


def _epilogue(acc, w, n_rows):
    d = acc.shape[2]
    blk = 1000
    grid = n_rows // blk
    return pl.pallas_call(
        _epilogue_body,
        grid=(grid,),
        in_specs=[
            pl.BlockSpec(memory_space=pltpu.SMEM),
            pl.BlockSpec((2, blk, d), lambda i: (0, i, 0)),
        ],
        out_specs=pl.BlockSpec((blk, d), lambda i: (i, 0)),
        out_shape=jax.ShapeDtypeStruct((n_rows, d), jnp.float32),
    )(w, acc)


def kernel(x, weight, adj_rows, adj_cols, adj_vals, idx):
    rows = lax.dynamic_index_in_dim(adj_rows, idx, 0, keepdims=False)
    cols = lax.dynamic_index_in_dim(adj_cols, idx, 0, keepdims=False)
    vals = lax.dynamic_index_in_dim(adj_vals, idx, 0, keepdims=False)
    w = lax.dynamic_index_in_dim(weight, idx, 0, keepdims=False)

    e = rows.shape[0]
    n = x.shape[0]
    n_chunks = e // _CHUNK
    n_main = n_chunks // _NW
    n_extra = n_chunks % _NW

    cv_f = jnp.concatenate(
        [cols.reshape(n_chunks, 1, _CHUNK).astype(jnp.int32),
         lax.bitcast_convert_type(vals.reshape(n_chunks, 1, _CHUNK)
                                  .astype(jnp.float32), jnp.int32)],
        axis=2)
    rows_f = rows.reshape(n_chunks, 1, _CHUNK).astype(jnp.int32)

    acc = _sc_spmm(x.astype(jnp.float32), cv_f, rows_f, n_main, n_extra)
    return _epilogue(acc, w.reshape(1).astype(jnp.float32), n)


# speedup vs baseline: 1.0044x; 1.0044x over previous
import functools

import jax
import jax.numpy as jnp
import numpy as np
from jax import lax
from jax.experimental import pallas as pl
from jax.experimental.pallas import tpu as pltpu
from jax.experimental.pallas import tpu_sc as plsc

_NC = 2
_NS = 16
_NW = _NC * _NS
_L = 16
_CHUNK = 128
_FLUSH = 80


def _sc_spmm_body(x_hbm, cv_f, rows_f, out_hbm,
                  b0, b1, b2, c0, c1, c2, r0, r1, r2,
                  acc_sh,
                  gs0, gs1, gs2, ss0, ss1, ss2, cs0, cs1, cs2,
                  rs0, rs1, rs2,
                  *, n_rows, n_main, n_extra):
    cid = lax.axis_index("c")
    sid = lax.axis_index("s")
    wid = sid * _NC + cid

    d = x_hbm.shape[1]
    buf = (b0, b1, b2)
    cbuf = (c0, c1, c2)
    rbuf = (r0, r1, r2)
    gsem = (gs0, gs1, gs2)
    ssem = (ss0, ss1, ss2)
    csem = (cs0, cs1, cs2)
    rsem = (rs0, rs1, rs2)

    n_units = n_rows // _FLUSH
    per = n_units // _NS
    hi = n_units % _NS
    ubase = sid * per + jnp.minimum(sid, hi)
    ucnt = per + jnp.where(sid < hi, 1, 0)

    def chunk_id(c):
        return wid * n_main + c

    def start_cv(s, c):
        pltpu.async_copy(cv_f.at[chunk_id(c)], cbuf[s], csem[s])

    def wait_cv(s, c):
        pltpu.make_async_copy(cv_f.at[chunk_id(c)], cbuf[s], csem[s]).wait()

    def start_rows(s, c):
        pltpu.async_copy(rows_f.at[chunk_id(c)], rbuf[s], rsem[s])

    def wait_rows(s, c):
        pltpu.make_async_copy(rows_f.at[chunk_id(c)], rbuf[s],
                              rsem[s]).wait()

    def start_gather(s):
        pltpu.async_copy(x_hbm.at[cbuf[s].at[0, pl.ds(0, _CHUNK)]], buf[s],
                         gsem[s])

    def wait_gather(s):
        pltpu.make_async_copy(x_hbm.at[cbuf[s].at[0, pl.ds(0, _CHUNK)]],
                              buf[s], gsem[s]).wait()

    def start_scatter(s):
        pltpu.async_copy(buf[s], acc_sh.at[rbuf[s].at[0]], ssem[s], add=True)

    def wait_scatter(s):
        pltpu.make_async_copy(buf[s], acc_sh.at[rbuf[s].at[0]],
                              ssem[s]).wait()

    def _scale(b, vref):
        @pl.loop(0, _CHUNK // _L)
        def _(g):
            v16i = vref[0, pl.ds(_CHUNK + g * _L, _L)]
            v16 = lax.bitcast_convert_type(v16i, jnp.float32)

            @pl.loop(0, _L, unroll=4)
            def _(t):
                vb = v16.at[jnp.full((_L,), t, jnp.int32)].get(
                    mode="promise_in_bounds")
                e = g * _L + t
                for j in range(d // _L):
                    sl = pl.ds(j * _L, _L)
                    b[e, sl] = b[e, sl] * vb

    n3 = (n_main // 3) * 3

    if n3 >= 3:
        start_cv(0, 0)
        start_rows(0, 0)
        start_cv(1, 1)
        start_rows(1, 1)
        start_cv(2, 2)
        wait_cv(0, 0)
        start_gather(0)
        wait_cv(1, 1)
        start_gather(1)

    @pl.loop(0, _FLUSH)
    def _zero_buf(r):
        for j in range(d // _L):
            b2[r, pl.ds(j * _L, _L)] = jnp.zeros((_L,), jnp.float32)

    @pl.loop(ubase, ubase + ucnt)
    def _zero(u):
        off = pl.multiple_of(u * _FLUSH, 8)
        pltpu.sync_copy(b2.at[pl.ds(0, _FLUSH)], acc_sh.at[pl.ds(off, _FLUSH)])
    plsc.subcore_barrier()

    if n3 >= 3:
        @pl.loop(0, n3, step=3)
        def _main(c):
            for k in range(3):
                cc = c + k
                s = k
                s2 = (k + 2) % 3

                @pl.when(cc >= 1)
                def _():
                    wait_scatter(s2)

                @pl.when(cc + 2 < n3)
                def _():
                    start_rows(s2, cc + 2)
                    wait_cv(s2, cc + 2)
                    start_gather(s2)

                wait_gather(s)
                _scale(buf[s], cbuf[s])

                @pl.when(cc + 3 < n3)
                def _():
                    start_cv(s, cc + 3)

                wait_rows(s, cc)
                start_scatter(s)

        wait_scatter((n3 - 1) % 3)

    for cc in range(n3, n_main):
        start_cv(0, cc)
        start_rows(0, cc)
        wait_cv(0, cc)
        start_gather(0)
        wait_gather(0)
        _scale(b0, c0)
        wait_rows(0, cc)
        start_scatter(0)
        wait_scatter(0)

    if n_extra:
        @pl.when(wid < n_extra)
        def _tail():
            ct = _NW * n_main + wid
            pltpu.async_copy(cv_f.at[ct], c0, cs0)
            pltpu.async_copy(rows_f.at[ct], r0, rs0)
            pltpu.make_async_copy(cv_f.at[ct], c0, cs0).wait()
            pltpu.make_async_copy(rows_f.at[ct], r0, rs0).wait()
            pltpu.sync_copy(x_hbm.at[c0.at[0, pl.ds(0, _CHUNK)]], b0)
            _scale(b0, c0)
            pltpu.sync_copy(b0, acc_sh.at[r0.at[0]], add=True)

    plsc.subcore_barrier()

    @pl.loop(ubase, ubase + ucnt)
    def _flush(u):
        off = pl.multiple_of(u * _FLUSH, 8)
        pltpu.sync_copy(acc_sh.at[pl.ds(off, _FLUSH)],
                        out_hbm.at[cid, pl.ds(off, _FLUSH)])


def _sc_spmm(x, cv_f, rows_f, n_main, n_extra):
    n_rows, d = x.shape

    body = functools.partial(
        _sc_spmm_body, n_rows=n_rows, n_main=n_main, n_extra=n_extra)
    dma = pltpu.SemaphoreType.DMA
    return pl.kernel(
        body,
        out_type=jax.ShapeDtypeStruct((_NC, n_rows, d), jnp.float32),
        mesh=plsc.VectorSubcoreMesh(core_axis_name="c", subcore_axis_name="s"),
        scratch_types=[
            pltpu.VMEM((_CHUNK, d), jnp.float32),
            pltpu.VMEM((_CHUNK, d), jnp.float32),
            pltpu.VMEM((_CHUNK, d), jnp.float32),
            pltpu.VMEM((1, 2 * _CHUNK), jnp.int32),
            pltpu.VMEM((1, 2 * _CHUNK), jnp.int32),
            pltpu.VMEM((1, 2 * _CHUNK), jnp.int32),
            pltpu.VMEM((1, _CHUNK), jnp.int32),
            pltpu.VMEM((1, _CHUNK), jnp.int32),
            pltpu.VMEM((1, _CHUNK), jnp.int32),
            pltpu.VMEM_SHARED((n_rows, d), jnp.float32),
            dma, dma, dma,
            dma, dma, dma,
            dma, dma, dma,
            dma, dma, dma,
        ],
    )(x, cv_f, rows_f)


def _epilogue_body(w_ref, acc_ref, o_ref):
    a = acc_ref[0] + acc_ref[1]
    s = a * w_ref[0]
    n2 = jnp.sum(s * s, axis=1, keepdims=True)
    y = s * lax.rsqrt(jnp.maximum(n2, 1e-24))
    o_ref[...] = 0.5 * y * (1.0 + lax.erf(y * np.float32(1.0 / np.sqrt(2.0))))


def _epilogue(acc, w, n_rows):
    d = acc.shape[2]
    blk = 1000
    grid = n_rows // blk
    return pl.pallas_call(
        _epilogue_body,
        grid=(grid,),
        in_specs=[
            pl.BlockSpec(memory_space=pltpu.SMEM),
            pl.BlockSpec((2, blk, d), lambda i: (0, i, 0)),
        ],
        out_specs=pl.BlockSpec((blk, d), lambda i: (i, 0)),
        out_shape=jax.ShapeDtypeStruct((n_rows, d), jnp.float32),
    )(w, acc)


def kernel(x, weight, adj_rows, adj_cols, adj_vals, idx):
    rows = lax.dynamic_index_in_dim(adj_rows, idx, 0, keepdims=False)
    cols = lax.dynamic_index_in_dim(adj_cols, idx, 0, keepdims=False)
    vals = lax.dynamic_index_in_dim(adj_vals, idx, 0, keepdims=False)
    w = lax.dynamic_index_in_dim(weight, idx, 0, keepdims=False)

    e = rows.shape[0]
    n = x.shape[0]
    n_chunks = e // _CHUNK
    n_main = n_chunks // _NW
    n_extra = n_chunks % _NW

    cv_f = jnp.concatenate(
        [cols.reshape(n_chunks, 1, _CHUNK).astype(jnp.int32),
         lax.bitcast_convert_type(vals.reshape(n_chunks, 1, _CHUNK)
                                  .astype(jnp.float32), jnp.int32)],
        axis=2)
    rows_f = rows.reshape(n_chunks, 1, _CHUNK).astype(jnp.int32)

    acc = _sc_spmm(x.astype(jnp.float32), cv_f, rows_f, n_main, n_extra)
    return _epilogue(acc, w.reshape(1).astype(jnp.float32), n)
